# Initial kernel scaffold; baseline (speedup 1.0000x reference)
import jax
import jax.numpy as jnp
from jax.experimental import pallas as pl


def kernel(hidden_states, input_layernorm, post_attention_layernorm, W_qkv, W_o, W_router, gate_up_proj, down_proj):
    raise NotImplementedError("write your pallas kernel here")


# speedup vs baseline: 1.0300x; 1.0300x over previous
import math

import jax
import jax.numpy as jnp
from jax.experimental import pallas as pl
from jax.experimental.pallas import tpu as pltpu

B, S = 1, 2048
H = 768
NH = 12
NKV = 12
HD = 64
I = 2048
E = 8
T = B * S

BT = 256
NTB = T // BT


def _qkv_body(x_ref, ln_ref, w_ref, qkv_ref):
    x = x_ref[...]
    var = jnp.mean(x * x, axis=-1, keepdims=True)
    xn = (x * jax.lax.rsqrt(var + 1e-6)) * ln_ref[...]
    qkv_ref[...] = jax.lax.dot_general(
        xn.astype(jnp.bfloat16), w_ref[...],
        (((1,), (0,)), ((), ())), preferred_element_type=jnp.float32
    ).astype(jnp.bfloat16)


def _attn_body(q_ref, k_ref, v_ref, o_ref):
    q = q_ref[0]
    k = k_ref[0]
    s = jax.lax.dot_general(
        q, k, (((1,), (1,)), ((), ())), preferred_element_type=jnp.float32
    ) * (1.0 / math.sqrt(HD))
    m = jnp.max(s, axis=-1, keepdims=True)
    p = jnp.exp(s - m)
    denom = jnp.sum(p, axis=-1, keepdims=True)
    o = jax.lax.dot_general(
        p.astype(jnp.bfloat16), v_ref[0],
        (((1,), (0,)), ((), ())), preferred_element_type=jnp.float32)
    o_ref[0] = (o / denom).astype(jnp.bfloat16)


def _post_body(ao_ref, wo_ref, res_ref, ln2_ref, wr_ref,
               hid_ref, hf_ref, cmb_ref):
    attn_out = jax.lax.dot_general(
        ao_ref[...], wo_ref[...],
        (((1,), (0,)), ((), ())), preferred_element_type=jnp.float32)
    hidden = res_ref[...] + attn_out
    hid_ref[...] = hidden
    var = jnp.mean(hidden * hidden, axis=-1, keepdims=True)
    hf = hidden * jax.lax.rsqrt(var + 1e-6) * ln2_ref[...]
    hf_ref[...] = hf.astype(jnp.bfloat16)
    logits = jax.lax.dot_general(
        hf, wr_ref[...], (((1,), (0,)), ((), ())),
        preferred_element_type=jnp.float32)
    iota = jax.lax.broadcasted_iota(jnp.int32, logits.shape, 1)
    m0 = jnp.max(logits, axis=-1, keepdims=True)
    i0 = jnp.min(jnp.where(logits >= m0, iota, E), axis=-1, keepdims=True)
    l2 = jnp.where(iota == i0, -jnp.inf, logits)
    m1 = jnp.max(l2, axis=-1, keepdims=True)
    i1 = jnp.min(jnp.where(l2 >= m1, iota, E), axis=-1, keepdims=True)
    a = jnp.exp(m1 - m0)
    w0 = 1.0 / (1.0 + a)
    w1 = a / (1.0 + a)
    cmb_ref[...] = (jnp.where(iota == i0, w0, 0.0)
                    + jnp.where(iota == i1, w1, 0.0))


def _moe_body(hf_ref, cmb_ref, hid_ref, gup_ref, down_ref, out_ref):
    e = pl.program_id(1)
    x = hf_ref[...]
    gu = jax.lax.dot_general(
        x, gup_ref[0], (((1,), (0,)), ((), ())),
        preferred_element_type=jnp.float32)
    g = gu[:, :I]
    u = gu[:, I:]
    act = (g * jax.nn.sigmoid(g) * u).astype(jnp.bfloat16)
    y = jax.lax.dot_general(
        act, down_ref[0], (((1,), (0,)), ((), ())),
        preferred_element_type=jnp.float32)
    sel = (jax.lax.broadcasted_iota(jnp.int32, (BT, E), 1) == e)
    w = jnp.sum(cmb_ref[...] * sel.astype(jnp.float32), axis=-1, keepdims=True)

    @pl.when(e == 0)
    def _():
        out_ref[...] = hid_ref[...] + w * y

    @pl.when(e > 0)
    def _():
        out_ref[...] = out_ref[...] + w * y


def kernel(hidden_states, input_layernorm, post_attention_layernorm,
           W_qkv, W_o, W_router, gate_up_proj, down_proj):
    x = hidden_states.reshape(T, H)
    wqkv = W_qkv.astype(jnp.bfloat16)
    wo = W_o.astype(jnp.bfloat16)
    gup = gate_up_proj.astype(jnp.bfloat16)
    down = down_proj.astype(jnp.bfloat16)

    qkv = pl.pallas_call(
        _qkv_body,
        grid=(NTB,),
        in_specs=[
            pl.BlockSpec((BT, H), lambda i: (i, 0)),
            pl.BlockSpec((H,), lambda i: (0,)),
            pl.BlockSpec((H, (NH + 2 * NKV) * HD), lambda i: (0, 0)),
        ],
        out_specs=pl.BlockSpec((BT, (NH + 2 * NKV) * HD), lambda i: (i, 0)),
        out_shape=jax.ShapeDtypeStruct((T, (NH + 2 * NKV) * HD), jnp.bfloat16),
    )(x, input_layernorm, wqkv)

    q = qkv[:, :NH * HD].reshape(T, NH, HD).transpose(1, 0, 2)
    k = qkv[:, NH * HD:(NH + NKV) * HD].reshape(T, NKV, HD).transpose(1, 0, 2)
    v = qkv[:, (NH + NKV) * HD:].reshape(T, NKV, HD).transpose(1, 0, 2)

    attn_heads = pl.pallas_call(
        _attn_body,
        grid=(NH, NTB),
        in_specs=[
            pl.BlockSpec((1, BT, HD), lambda h, i: (h, i, 0)),
            pl.BlockSpec((1, T, HD), lambda h, i: (h, 0, 0)),
            pl.BlockSpec((1, T, HD), lambda h, i: (h, 0, 0)),
        ],
        out_specs=pl.BlockSpec((1, BT, HD), lambda h, i: (h, i, 0)),
        out_shape=jax.ShapeDtypeStruct((NH, T, HD), jnp.bfloat16),
    )(q, k, v)
    attn_out = attn_heads.transpose(1, 0, 2).reshape(T, NH * HD)

    hidden, hf, combine = pl.pallas_call(
        _post_body,
        grid=(NTB,),
        in_specs=[
            pl.BlockSpec((BT, NH * HD), lambda i: (i, 0)),
            pl.BlockSpec((NH * HD, H), lambda i: (0, 0)),
            pl.BlockSpec((BT, H), lambda i: (i, 0)),
            pl.BlockSpec((H,), lambda i: (0,)),
            pl.BlockSpec((H, E), lambda i: (0, 0)),
        ],
        out_specs=(
            pl.BlockSpec((BT, H), lambda i: (i, 0)),
            pl.BlockSpec((BT, H), lambda i: (i, 0)),
            pl.BlockSpec((BT, E), lambda i: (i, 0)),
        ),
        out_shape=(
            jax.ShapeDtypeStruct((T, H), jnp.float32),
            jax.ShapeDtypeStruct((T, H), jnp.bfloat16),
            jax.ShapeDtypeStruct((T, E), jnp.float32),
        ),
    )(attn_out, wo, x, post_attention_layernorm, W_router)

    out = pl.pallas_call(
        _moe_body,
        grid=(NTB, E),
        in_specs=[
            pl.BlockSpec((BT, H), lambda i, e: (i, 0)),
            pl.BlockSpec((BT, E), lambda i, e: (i, 0)),
            pl.BlockSpec((BT, H), lambda i, e: (i, 0)),
            pl.BlockSpec((1, H, 2 * I), lambda i, e: (e, 0, 0)),
            pl.BlockSpec((1, I, H), lambda i, e: (e, 0, 0)),
        ],
        out_specs=pl.BlockSpec((BT, H), lambda i, e: (i, 0)),
        out_shape=jax.ShapeDtypeStruct((T, H), jnp.float32),
    )(hf, combine, hidden, gup, down)

    return out.reshape(B, S, H)
